# Initial kernel scaffold; baseline (speedup 1.0000x reference)
import jax
import jax.numpy as jnp
from jax.experimental import pallas as pl


def kernel(context, target, emb, W):
    raise NotImplementedError("write your pallas kernel here")


# speedup vs baseline: 2.5102x; 2.5102x over previous
import functools

import jax
import jax.numpy as jnp
from jax import lax
from jax.experimental import pallas as pl
from jax.experimental.pallas import tpu as pltpu
from jax.experimental.pallas import tpu_sc as plsc

VOCAB = 100000
D = 64
B = 1024
NCTX = 10

NC, NS = 2, 16
NW = NC * NS
CTX_PER_W = (B * NCTX) // NW
TGT_PER_W = B // NW

VT = 2000
NSTEPS = VOCAB // VT


@functools.cache
def _make_sc_gather():
    mesh = plsc.VectorSubcoreMesh(core_axis_name="c", subcore_axis_name="s")

    @functools.partial(
        pl.kernel,
        mesh=mesh,
        out_type=(
            jax.ShapeDtypeStruct((B * NCTX, 2 * D), jnp.float32),
            jax.ShapeDtypeStruct((B, 2 * D), jnp.float32),
        ),
        scratch_types=[
            pltpu.VMEM((CTX_PER_W,), jnp.int32),
            pltpu.VMEM((CTX_PER_W, 2 * D), jnp.float32),
            pltpu.VMEM((TGT_PER_W,), jnp.int32),
            pltpu.VMEM((TGT_PER_W, 2 * D), jnp.float32),
            pltpu.SemaphoreType.DMA,
        ],
    )
    def sc_gather(emb_hbm, w_hbm, cidx_hbm, tidx_hbm, ctx_out, wt_out,
                  cidx_v, crows_v, tidx_v, trows_v, sem):
        wid = lax.axis_index("s") * NC + lax.axis_index("c")
        cbase = wid * CTX_PER_W
        pltpu.sync_copy(cidx_hbm.at[pl.ds(cbase, CTX_PER_W)], cidx_v)
        pltpu.async_copy(emb_hbm.at[cidx_v], crows_v, sem).wait()
        pltpu.sync_copy(crows_v, ctx_out.at[pl.ds(cbase, CTX_PER_W)])
        tbase = wid * TGT_PER_W
        pltpu.sync_copy(tidx_hbm.at[pl.ds(tbase, TGT_PER_W)], tidx_v)
        pltpu.async_copy(w_hbm.at[tidx_v], trows_v, sem).wait()
        pltpu.sync_copy(trows_v, wt_out.at[pl.ds(tbase, TGT_PER_W)])

    return sc_gather


def _half(g, p):
    lo, hi = g[:, :D], g[:, D:]
    return lo + p * (hi - lo)


def _tc_body(ctx_ref, cpar_ref, wt_ref, tpar_ref, w_ref, nll_ref,
             cm_ref, tl_ref, s_ref):
    i = pl.program_id(0)

    @pl.when(i == 0)
    def _init():
        acc = _half(ctx_ref[0], cpar_ref[0])
        for j in range(1, NCTX):
            acc = acc + _half(ctx_ref[j], cpar_ref[j])
        cm = acc * (1.0 / NCTX)
        cm_ref[...] = cm
        wt = _half(wt_ref[...], tpar_ref[...])
        tl_ref[...] = jnp.sum(cm * wt, axis=1, keepdims=True)
        s_ref[...] = jnp.zeros_like(s_ref)

    cm16 = cm_ref[...].astype(jnp.bfloat16)
    w16 = w_ref[...].astype(jnp.bfloat16)
    logits = lax.dot_general(
        cm16, w16, (((1,), (1,)), ((), ())),
        preferred_element_type=jnp.float32,
    )
    s_ref[...] += jnp.sum(jnp.exp(logits), axis=1, keepdims=True)

    @pl.when(i == NSTEPS - 1)
    def _fini():
        nll_ref[...] = jnp.log(s_ref[...]) - tl_ref[...]


def _tc_nll(ctxg, cpar, wt, tpar, W, interpret=False):
    return pl.pallas_call(
        _tc_body,
        grid=(NSTEPS,),
        in_specs=[
            pl.BlockSpec((NCTX, B, 2 * D), lambda i: (0, 0, 0)),
            pl.BlockSpec((NCTX, B, 1), lambda i: (0, 0, 0)),
            pl.BlockSpec((B, 2 * D), lambda i: (0, 0)),
            pl.BlockSpec((B, 1), lambda i: (0, 0)),
            pl.BlockSpec((VT, D), lambda i: (i, 0)),
        ],
        out_specs=pl.BlockSpec((B, 1), lambda i: (0, 0)),
        out_shape=jax.ShapeDtypeStruct((B, 1), jnp.float32),
        scratch_shapes=[
            pltpu.VMEM((B, D), jnp.float32),
            pltpu.VMEM((B, 1), jnp.float32),
            pltpu.VMEM((B, 1), jnp.float32),
        ],
        interpret=interpret,
    )(ctxg, cpar, wt, tpar, W)


def kernel(context, target, emb, W):
    embp = emb.reshape(VOCAB // 2, 2 * D)
    wp = W.reshape(VOCAB // 2, 2 * D)
    cidx = context.astype(jnp.int32).T.reshape(-1)
    tidx = target.astype(jnp.int32)
    cpar = (cidx & 1).astype(jnp.float32).reshape(NCTX, B, 1)
    tpar = (tidx & 1).astype(jnp.float32)[:, None]
    ctxg, wt = _make_sc_gather()(embp, wp, cidx >> 1, tidx >> 1)
    nll = _tc_nll(ctxg.reshape(NCTX, B, 2 * D), cpar, wt, tpar, W)
    return jnp.mean(nll)
